# baseline (speedup 1.0000x reference)
import functools
import math

import numpy as np
import jax
import jax.numpy as jnp
from jax import lax
from jax.experimental import pallas as pl
from jax.experimental.pallas import tpu as pltpu
from jax.experimental.pallas import tpu_sc as plsc

_NSPH = 7
_NRAD = 6
_CUTOFF = 5.0
_ENV_P = 6
_NE = 160000
_NQ = 320000
_NB = 48
_NF = _NSPH * _NSPH
_NO = _NF * _NRAD


def _bess_jn_np(x, n):
    x = np.asarray(x, dtype=np.float64)
    j0 = np.sin(x) / x
    if n == 0:
        return j0
    j1 = np.sin(x) / x**2 - np.cos(x) / x
    if n == 1:
        return j1
    jm1, jc = j0, j1
    for l in range(1, n):
        jm1, jc = jc, (2 * l + 1) / x * jc - jm1
    return jc


def _bess_zeros_np(n, k):
    zerosj = np.zeros((n, k))
    zerosj[0] = np.arange(1, k + 1) * np.pi
    points = np.arange(1, k + n) * np.pi
    racines = np.zeros(k + n - 1)
    for i in range(1, n):
        for j in range(k + n - 1 - i):
            a, b = float(points[j]), float(points[j + 1])
            fa = _bess_jn_np(a, i)
            for _ in range(200):
                m = 0.5 * (a + b)
                fm = _bess_jn_np(m, i)
                if fa * fm <= 0:
                    b = m
                else:
                    a, fa = m, fm
            racines[j] = 0.5 * (a + b)
        points = racines.copy()
        zerosj[i][:k] = racines[:k]
    return zerosj


def _sph_feats_np(theta, phi):
    L = _NSPH
    theta = np.asarray(theta, dtype=np.float64)
    phi = np.asarray(phi, dtype=np.float64)
    ct, st = np.cos(theta), np.sin(theta)
    P = [[None] * (l + 1) for l in range(L)]
    P[0][0] = np.ones_like(ct)
    P[1][0] = ct
    for l in range(2, L):
        P[l][0] = ((2 * l - 1) * ct * P[l - 1][0] - (l - 1) * P[l - 2][0]) / l
    for m in range(1, L):
        P[m][m] = (1 - 2 * m) * P[m - 1][m - 1]
        if m + 1 < L:
            P[m + 1][m] = (2 * m + 1) * ct * P[m][m]
        for l in range(m + 2, L):
            P[l][m] = ((2 * l - 1) * ct * P[l - 1][m] - (m + l - 1) * P[l - 2][m]) / (l - m)

    def pref(l, m):
        return ((2 * l + 1) * math.factorial(l - abs(m)) /
                (4 * math.pi * math.factorial(l + abs(m)))) ** 0.5

    C = [np.ones_like(theta)]
    S = [np.zeros_like(theta)]
    for m in range(1, L):
        C.append(st**m * np.cos(m * phi))
        S.append(st**m * np.sin(m * phi))
    feats = []
    for l in range(L):
        feats.append(pref(l, 0) * P[l][0])
        pos = [2**0.5 * pref(l, m) * C[m] * P[l][m] for m in range(1, l + 1)]
        neg = [2**0.5 * pref(l, -m) * S[m] * P[l][m] for m in range(1, l + 1)]
        feats.extend(pos)
        feats.extend(reversed(neg))
    return np.stack(feats, axis=1)


def _feature_meta():
    meta = []
    for l in range(_NSPH):
        meta.append((l, 0, False))
        for m in range(1, l + 1):
            meta.append((l, m, False))
        for m in range(l, 0, -1):
            meta.append((l, m, True))
    return meta


def _build_constants():
    zeros = _bess_zeros_np(_NSPH, _NRAD)
    norm = np.array([[1.0 / np.sqrt(0.5 * _bess_jn_np(zeros[l, i], l + 1) ** 2)
                      for i in range(_NRAD)] for l in range(_NSPH)])
    norm_const = (1.0 / _CUTOFF) ** 1.5

    c1 = np.zeros((16, _NB), dtype=np.float64)
    c1[0, :] = 1.0
    for l in range(_NSPH):
        for j in range(_NRAD):
            col = 6 * l + j
            c1[0, col] = zeros[l, j]
            c1[1, col] = norm[l, j] * norm_const
            c1[2 + l, col] = 1.0

    meta = _feature_meta()
    s_alpha = np.linspace(0.013, math.pi - 0.013, 41)
    design = np.concatenate(
        [np.cos(np.outer(s_alpha, np.arange(7)))] +
        [np.sin(np.outer(s_alpha, np.arange(1, 7)))], axis=1)
    acoef = np.zeros((13, _NF))
    feats_at = {0.0: _sph_feats_np(s_alpha, 0.0)}
    for k, (l, m, is_sin) in enumerate(meta):
        phi0 = (math.pi / (2 * m)) if is_sin else 0.0
        if phi0 not in feats_at:
            feats_at[phi0] = _sph_feats_np(s_alpha, phi0)
        avals = feats_at[phi0][:, k]
        acoef[:, k] = np.linalg.lstsq(design, avals, rcond=None)[0]

    jmp = np.zeros((8, 16), dtype=np.float64)
    jmp[0, 0:7] = np.arange(7)
    jmp[0, 7:13] = np.arange(1, 7)
    jmp[1, 7:13] = -math.pi / 2

    u16 = np.zeros((16, _NO), dtype=np.float64)
    mb = np.zeros((8, _NO), dtype=np.float64)
    a48 = np.zeros((_NB, _NO), dtype=np.float64)
    for c in range(_NO):
        k, j = divmod(c, 6)
        l, m, is_sin = meta[k]
        u16[0:13, c] = acoef[:, k]
        mb[0, c] = m
        mb[1, c] = -math.pi / 2 if is_sin else 0.0
        a48[6 * l + j, c] = 1.0

    f32 = np.float32
    return (c1.astype(f32), jmp.astype(f32), u16.astype(f32),
            mb.astype(f32), a48.astype(f32))


_C1, _JMP, _U16, _MB, _A48 = _build_constants()


_BE = 2000


def _rbf_table_body(d_ref, c1_ref, o_ref):
    d = d_ref[...] * (1.0 / _CUTOFF)
    zr = c1_ref[0:1, :]
    nr = c1_ref[1:2, :]
    z = d * zr
    s = jnp.sin(z)
    c = jnp.cos(z)
    rz = 1.0 / z
    j0 = s * rz
    j1 = (s * rz - c) * rz
    res = j0 * c1_ref[2:3, :] + j1 * c1_ref[3:4, :]
    jm1, jc = j0, j1
    for l in range(1, _NSPH - 1):
        jm1, jc = jc, (2 * l + 1) * rz * jc - jm1
        res = res + jc * c1_ref[4 + l - 1:5 + l - 1, :]
    p = _ENV_P
    a = -(p + 1) * (p + 2) / 2.0
    b = float(p * (p + 2))
    cc = -p * (p + 1) / 2.0
    d2 = d * d
    d4 = d2 * d2
    dpm1 = d4 * d
    env = 1.0 / d + a * dpm1 + b * dpm1 * d + cc * dpm1 * d2
    env = jnp.where(d < 1.0, env, jnp.zeros_like(d))
    o_ref[...] = res * nr * env


def _rbf_table(d_col):
    grid = _NE // _BE
    return pl.pallas_call(
        _rbf_table_body,
        grid=(grid,),
        in_specs=[
            pl.BlockSpec((_BE, 1), lambda i: (i, 0)),
            pl.BlockSpec((16, _NB), lambda i: (0, 0)),
        ],
        out_specs=pl.BlockSpec((_BE, _NB), lambda i: (i, 0)),
        out_shape=jax.ShapeDtypeStruct((_NE, _NB), jnp.float32),
        compiler_params=pltpu.CompilerParams(
            dimension_semantics=("arbitrary",)),
    )(d_col, jnp.asarray(_C1))


_NC = 2
_NS = 16
_NW = _NC * _NS
_CH = 128
_NCHUNKS = _NQ // _CH


def _sc_gather(table, idx):
    idx3 = idx.reshape(_NCHUNKS, 1, _CH)

    @functools.partial(
        pl.kernel,
        mesh=plsc.VectorSubcoreMesh(core_axis_name="c", subcore_axis_name="s"),
        out_type=jax.ShapeDtypeStruct((_NQ, _NB), jnp.float32),
        scratch_types=[
            pltpu.VMEM((1, _CH), jnp.int32),
            pltpu.VMEM((_CH, _NB), jnp.float32),
            pltpu.SemaphoreType.DMA,
        ],
        compiler_params=pltpu.CompilerParams(use_tc_tiling_on_sc=False),
    )
    def gk(tab_ref, idx_ref, out_ref, idx_v, rows_v, sem):
        wid = lax.axis_index("s") * _NC + lax.axis_index("c")
        nt = (_NCHUNKS - wid + _NW - 1) // _NW

        def body(t, carry):
            chunk = wid + t * _NW
            pltpu.sync_copy(idx_ref.at[chunk], idx_v)
            pltpu.async_copy(tab_ref.at[idx_v.at[0]], rows_v, sem).wait()
            pltpu.sync_copy(rows_v, out_ref.at[pl.ds(chunk * _CH, _CH)])
            return carry

        lax.fori_loop(0, nt, body, 0)

    return gk(table, idx3)


_BQ = 2000


def _combine_body(g_ref, a_ref, t_ref, jmp_ref, u_ref, mb_ref, a48_ref, o_ref):
    al = a_ref[...]
    th = t_ref[...]
    ca = jnp.cos(al * jmp_ref[0:1, :] + jmp_ref[1:2, :])
    ap = jnp.dot(ca, u_ref[...], preferred_element_type=jnp.float32)
    trg = jnp.cos(th * mb_ref[0:1, :] + mb_ref[1:2, :])
    g = jnp.dot(g_ref[...], a48_ref[...], preferred_element_type=jnp.float32)
    o_ref[...] = ap * trg * g


def _combine(g, al_col, th_col):
    grid = _NQ // _BQ
    return pl.pallas_call(
        _combine_body,
        grid=(grid,),
        in_specs=[
            pl.BlockSpec((_BQ, _NB), lambda i: (i, 0)),
            pl.BlockSpec((_BQ, 1), lambda i: (i, 0)),
            pl.BlockSpec((_BQ, 1), lambda i: (i, 0)),
            pl.BlockSpec((8, 16), lambda i: (0, 0)),
            pl.BlockSpec((16, _NO), lambda i: (0, 0)),
            pl.BlockSpec((8, _NO), lambda i: (0, 0)),
            pl.BlockSpec((_NB, _NO), lambda i: (0, 0)),
        ],
        out_specs=pl.BlockSpec((_BQ, _NO), lambda i: (i, 0)),
        out_shape=jax.ShapeDtypeStruct((_NQ, _NO), jnp.float32),
        compiler_params=pltpu.CompilerParams(
            dimension_semantics=("arbitrary",)),
    )(g, al_col, th_col, jnp.asarray(_JMP), jnp.asarray(_U16),
      jnp.asarray(_MB), jnp.asarray(_A48))


def kernel(D_ca, Alpha_cab, Theta_cabd, id4_reduce_ca, Kidx):
    table = _rbf_table(D_ca.reshape(_NE, 1))
    g = _sc_gather(table, id4_reduce_ca)
    return _combine(g, Alpha_cab.reshape(_NQ, 1), Theta_cabd.reshape(_NQ, 1))


# speedup vs baseline: 1.3761x; 1.3761x over previous
import functools
import math

import numpy as np
import jax
import jax.numpy as jnp
from jax import lax
from jax.experimental import pallas as pl
from jax.experimental.pallas import tpu as pltpu
from jax.experimental.pallas import tpu_sc as plsc

_NSPH = 7
_NRAD = 6
_CUTOFF = 5.0
_ENV_P = 6
_NE = 160000
_NQ = 320000
_NB = 48
_NF = _NSPH * _NSPH
_NO = _NF * _NRAD


def _bess_jn_np(x, n):
    x = np.asarray(x, dtype=np.float64)
    j0 = np.sin(x) / x
    if n == 0:
        return j0
    j1 = np.sin(x) / x**2 - np.cos(x) / x
    if n == 1:
        return j1
    jm1, jc = j0, j1
    for l in range(1, n):
        jm1, jc = jc, (2 * l + 1) / x * jc - jm1
    return jc


def _bess_zeros_np(n, k):
    zerosj = np.zeros((n, k))
    zerosj[0] = np.arange(1, k + 1) * np.pi
    points = np.arange(1, k + n) * np.pi
    racines = np.zeros(k + n - 1)
    for i in range(1, n):
        for j in range(k + n - 1 - i):
            a, b = float(points[j]), float(points[j + 1])
            fa = _bess_jn_np(a, i)
            for _ in range(200):
                m = 0.5 * (a + b)
                fm = _bess_jn_np(m, i)
                if fa * fm <= 0:
                    b = m
                else:
                    a, fa = m, fm
            racines[j] = 0.5 * (a + b)
        points = racines.copy()
        zerosj[i][:k] = racines[:k]
    return zerosj


def _sph_feats_np(theta, phi):
    L = _NSPH
    theta = np.asarray(theta, dtype=np.float64)
    phi = np.asarray(phi, dtype=np.float64)
    ct, st = np.cos(theta), np.sin(theta)
    P = [[None] * (l + 1) for l in range(L)]
    P[0][0] = np.ones_like(ct)
    P[1][0] = ct
    for l in range(2, L):
        P[l][0] = ((2 * l - 1) * ct * P[l - 1][0] - (l - 1) * P[l - 2][0]) / l
    for m in range(1, L):
        P[m][m] = (1 - 2 * m) * P[m - 1][m - 1]
        if m + 1 < L:
            P[m + 1][m] = (2 * m + 1) * ct * P[m][m]
        for l in range(m + 2, L):
            P[l][m] = ((2 * l - 1) * ct * P[l - 1][m] - (m + l - 1) * P[l - 2][m]) / (l - m)

    def pref(l, m):
        return ((2 * l + 1) * math.factorial(l - abs(m)) /
                (4 * math.pi * math.factorial(l + abs(m)))) ** 0.5

    C = [np.ones_like(theta)]
    S = [np.zeros_like(theta)]
    for m in range(1, L):
        C.append(st**m * np.cos(m * phi))
        S.append(st**m * np.sin(m * phi))
    feats = []
    for l in range(L):
        feats.append(pref(l, 0) * P[l][0])
        pos = [2**0.5 * pref(l, m) * C[m] * P[l][m] for m in range(1, l + 1)]
        neg = [2**0.5 * pref(l, -m) * S[m] * P[l][m] for m in range(1, l + 1)]
        feats.extend(pos)
        feats.extend(reversed(neg))
    return np.stack(feats, axis=1)


def _feature_meta():
    meta = []
    for l in range(_NSPH):
        meta.append((l, 0, False))
        for m in range(1, l + 1):
            meta.append((l, m, False))
        for m in range(l, 0, -1):
            meta.append((l, m, True))
    return meta


def _build_constants():
    zeros = _bess_zeros_np(_NSPH, _NRAD)
    norm = np.array([[1.0 / np.sqrt(0.5 * _bess_jn_np(zeros[l, i], l + 1) ** 2)
                      for i in range(_NRAD)] for l in range(_NSPH)])
    norm_const = (1.0 / _CUTOFF) ** 1.5

    c1 = np.zeros((16, _NB), dtype=np.float64)
    c1[0, :] = 1.0
    for l in range(_NSPH):
        for j in range(_NRAD):
            col = 6 * l + j
            c1[0, col] = zeros[l, j]
            c1[1, col] = norm[l, j] * norm_const
            c1[2 + l, col] = 1.0

    meta = _feature_meta()
    s_alpha = np.linspace(0.013, math.pi - 0.013, 41)
    design = np.concatenate(
        [np.cos(np.outer(s_alpha, np.arange(7)))] +
        [np.sin(np.outer(s_alpha, np.arange(1, 7)))], axis=1)
    acoef = np.zeros((13, _NF))
    feats_at = {0.0: _sph_feats_np(s_alpha, 0.0)}
    for k, (l, m, is_sin) in enumerate(meta):
        phi0 = (math.pi / (2 * m)) if is_sin else 0.0
        if phi0 not in feats_at:
            feats_at[phi0] = _sph_feats_np(s_alpha, phi0)
        avals = feats_at[phi0][:, k]
        acoef[:, k] = np.linalg.lstsq(design, avals, rcond=None)[0]

    jmp = np.zeros((8, 16), dtype=np.float64)
    jmp[0, 0:7] = np.arange(7)
    jmp[0, 7:13] = np.arange(1, 7)
    jmp[1, 7:13] = -math.pi / 2

    u16 = np.zeros((16, _NO), dtype=np.float64)
    oh16 = np.zeros((16, _NO), dtype=np.float64)
    a48 = np.zeros((_NB, _NO), dtype=np.float64)
    for c in range(_NO):
        k, j = divmod(c, 6)
        l, m, is_sin = meta[k]
        u16[0:13, c] = acoef[:, k]
        oh16[(6 + m) if is_sin else m, c] = 1.0
        a48[6 * l + j, c] = 1.0

    f32 = np.float32
    return (c1.astype(f32), jmp.astype(f32), u16.astype(f32),
            oh16.astype(f32), a48.astype(f32))


_C1, _JMP, _U16, _OH16, _A48 = _build_constants()


_BE = 2000


def _rbf_table_body(d_ref, c1_ref, o_ref):
    d = d_ref[...] * (1.0 / _CUTOFF)
    zr = c1_ref[0:1, :]
    nr = c1_ref[1:2, :]
    z = d * zr
    s = jnp.sin(z)
    c = jnp.cos(z)
    rz = 1.0 / z
    j0 = s * rz
    j1 = (s * rz - c) * rz
    res = j0 * c1_ref[2:3, :] + j1 * c1_ref[3:4, :]
    jm1, jc = j0, j1
    for l in range(1, _NSPH - 1):
        jm1, jc = jc, (2 * l + 1) * rz * jc - jm1
        res = res + jc * c1_ref[4 + l - 1:5 + l - 1, :]
    p = _ENV_P
    a = -(p + 1) * (p + 2) / 2.0
    b = float(p * (p + 2))
    cc = -p * (p + 1) / 2.0
    d2 = d * d
    d4 = d2 * d2
    dpm1 = d4 * d
    env = 1.0 / d + a * dpm1 + b * dpm1 * d + cc * dpm1 * d2
    env = jnp.where(d < 1.0, env, jnp.zeros_like(d))
    o_ref[...] = res * nr * env


def _rbf_table(d_col):
    grid = _NE // _BE
    return pl.pallas_call(
        _rbf_table_body,
        grid=(grid,),
        in_specs=[
            pl.BlockSpec((_BE, 1), lambda i: (i, 0)),
            pl.BlockSpec((16, _NB), lambda i: (0, 0)),
        ],
        out_specs=pl.BlockSpec((_BE, _NB), lambda i: (i, 0)),
        out_shape=jax.ShapeDtypeStruct((_NE, _NB), jnp.float32),
        compiler_params=pltpu.CompilerParams(
            dimension_semantics=("arbitrary",)),
    )(d_col, jnp.asarray(_C1))


_NC = 2
_NS = 16
_NW = _NC * _NS
_CH = 128
_NCHUNKS = _NQ // _CH


def _sc_gather(table, idx):
    idx3 = idx.reshape(_NCHUNKS, 1, _CH)

    @functools.partial(
        pl.kernel,
        mesh=plsc.VectorSubcoreMesh(core_axis_name="c", subcore_axis_name="s"),
        out_type=jax.ShapeDtypeStruct((_NQ, _NB), jnp.float32),
        scratch_types=[
            pltpu.VMEM((1, _CH), jnp.int32),
            pltpu.VMEM((_CH, _NB), jnp.float32),
            pltpu.SemaphoreType.DMA,
        ],
        compiler_params=pltpu.CompilerParams(use_tc_tiling_on_sc=False),
    )
    def gk(tab_ref, idx_ref, out_ref, idx_v, rows_v, sem):
        wid = lax.axis_index("s") * _NC + lax.axis_index("c")
        nt = (_NCHUNKS - wid + _NW - 1) // _NW

        def body(t, carry):
            chunk = wid + t * _NW
            pltpu.sync_copy(idx_ref.at[chunk], idx_v)
            pltpu.async_copy(tab_ref.at[idx_v.at[0]], rows_v, sem).wait()
            pltpu.sync_copy(rows_v, out_ref.at[pl.ds(chunk * _CH, _CH)])
            return carry

        lax.fori_loop(0, nt, body, 0)

    return gk(table, idx3)


_BQ = 2000


def _combine_body(g_ref, a_ref, t_ref, jmp_ref, u_ref, oh_ref, a48_ref, o_ref):
    al = a_ref[...]
    th = t_ref[...]
    freq = jmp_ref[0:1, :]
    phase = jmp_ref[1:2, :]
    ca = jnp.cos(al * freq + phase)
    cp = jnp.cos(th * freq + phase)
    ap = jnp.dot(ca, u_ref[...], preferred_element_type=jnp.float32)
    ph = jnp.dot(cp, oh_ref[...], preferred_element_type=jnp.float32)
    g = jnp.dot(g_ref[...], a48_ref[...], preferred_element_type=jnp.float32)
    o_ref[...] = ap * ph * g


def _combine(g, al_col, th_col):
    grid = _NQ // _BQ
    return pl.pallas_call(
        _combine_body,
        grid=(grid,),
        in_specs=[
            pl.BlockSpec((_BQ, _NB), lambda i: (i, 0)),
            pl.BlockSpec((_BQ, 1), lambda i: (i, 0)),
            pl.BlockSpec((_BQ, 1), lambda i: (i, 0)),
            pl.BlockSpec((8, 16), lambda i: (0, 0)),
            pl.BlockSpec((16, _NO), lambda i: (0, 0)),
            pl.BlockSpec((16, _NO), lambda i: (0, 0)),
            pl.BlockSpec((_NB, _NO), lambda i: (0, 0)),
        ],
        out_specs=pl.BlockSpec((_BQ, _NO), lambda i: (i, 0)),
        out_shape=jax.ShapeDtypeStruct((_NQ, _NO), jnp.float32),
        compiler_params=pltpu.CompilerParams(
            dimension_semantics=("arbitrary",)),
    )(g, al_col, th_col, jnp.asarray(_JMP), jnp.asarray(_U16),
      jnp.asarray(_OH16), jnp.asarray(_A48))


def kernel(D_ca, Alpha_cab, Theta_cabd, id4_reduce_ca, Kidx):
    table = _rbf_table(D_ca.reshape(_NE, 1))
    g = _sc_gather(table, id4_reduce_ca)
    return _combine(g, Alpha_cab.reshape(_NQ, 1), Theta_cabd.reshape(_NQ, 1))
